# Initial kernel scaffold; baseline (speedup 1.0000x reference)
import jax
import jax.numpy as jnp
from jax.experimental import pallas as pl


def kernel(x, table, gamma, beta):
    raise NotImplementedError("write your pallas kernel here")


# speedup vs baseline: 8.7042x; 8.7042x over previous
import functools

import jax
import jax.numpy as jnp
from jax import lax
from jax.experimental import pallas as pl
from jax.experimental.pallas import tpu as pltpu
from jax.experimental.pallas import tpu_sc as plsc

EPS = 1e-5


def _norm_body(tab_ref, gamma_ref, beta_ref, out_ref):
    xv = tab_ref[...]
    mean = jnp.mean(xv, axis=-1, keepdims=True)
    xc = xv - mean
    var = jnp.mean(xc * xc, axis=-1, keepdims=True)
    out_ref[...] = xc * lax.rsqrt(var + EPS) * gamma_ref[...] + beta_ref[...]


def _normalize_table(table, gamma, beta):
    V, D = table.shape
    R = 2000
    assert V % R == 0
    return pl.pallas_call(
        _norm_body,
        grid=(V // R,),
        in_specs=[
            pl.BlockSpec((R, D), lambda i: (i, 0)),
            pl.BlockSpec((1, D), lambda i: (0, 0)),
            pl.BlockSpec((1, D), lambda i: (0, 0)),
        ],
        out_specs=pl.BlockSpec((R, D), lambda i: (i, 0)),
        out_shape=jax.ShapeDtypeStruct((V, D), jnp.float32),
    )(table, gamma.reshape(1, D), beta.reshape(1, D))


@functools.lru_cache(maxsize=None)
def _make_gather(V, D, N):
    info = plsc.get_sparse_core_info()
    NC, NS = info.num_cores, info.num_subcores
    NW = NC * NS
    assert N % NW == 0
    per_w = N // NW
    C = 512
    assert per_w % C == 0
    n_chunks = per_w // C
    mesh = plsc.VectorSubcoreMesh(core_axis_name="c", subcore_axis_name="s")

    @functools.partial(
        pl.kernel,
        mesh=mesh,
        out_type=jax.ShapeDtypeStruct((N, D), jnp.float32),
        scratch_types=[
            pltpu.VMEM((C,), jnp.int32),
            pltpu.VMEM((C, D), jnp.float32),
            pltpu.SemaphoreType.DMA,
        ],
    )
    def gather_k(tab_hbm, idx_hbm, out_hbm, idx_v, rows_v, sem):
        wid = lax.axis_index("s") * NC + lax.axis_index("c")
        base = wid * per_w

        def body(i, carry):
            off = base + i * C
            pltpu.sync_copy(idx_hbm.at[pl.ds(off, C)], idx_v)
            pltpu.async_copy(tab_hbm.at[idx_v], rows_v, sem).wait()
            pltpu.sync_copy(rows_v, out_hbm.at[pl.ds(off, C)])
            return carry

        lax.fori_loop(0, n_chunks, body, 0)

    return gather_k


def kernel(x, table, gamma, beta):
    B, L = x.shape
    V, D = table.shape
    norm = _normalize_table(table, gamma, beta)
    flat = x.reshape(-1).astype(jnp.int32)
    out = _make_gather(V, D, B * L)(norm, flat)
    return out.reshape(B, L, D)


# speedup vs baseline: 9.4859x; 1.0898x over previous
import functools

import jax
import jax.numpy as jnp
from jax import lax
from jax.experimental import pallas as pl
from jax.experimental.pallas import tpu as pltpu
from jax.experimental.pallas import tpu_sc as plsc

EPS = 1e-5


def _norm_body(tab_ref, gamma_ref, beta_ref, out_ref):
    xv = tab_ref[...]
    mean = jnp.mean(xv, axis=-1, keepdims=True)
    xc = xv - mean
    var = jnp.mean(xc * xc, axis=-1, keepdims=True)
    out_ref[...] = xc * lax.rsqrt(var + EPS) * gamma_ref[...] + beta_ref[...]


def _normalize_table(table, gamma, beta):
    V, D = table.shape
    R = 2000
    assert V % R == 0
    return pl.pallas_call(
        _norm_body,
        grid=(V // R,),
        in_specs=[
            pl.BlockSpec((R, D), lambda i: (i, 0)),
            pl.BlockSpec((1, D), lambda i: (0, 0)),
            pl.BlockSpec((1, D), lambda i: (0, 0)),
        ],
        out_specs=pl.BlockSpec((R, D), lambda i: (i, 0)),
        out_shape=jax.ShapeDtypeStruct((V, D), jnp.float32),
    )(table, gamma.reshape(1, D), beta.reshape(1, D))


@functools.lru_cache(maxsize=None)
def _make_gather(V, D, N):
    info = plsc.get_sparse_core_info()
    NC, NS = info.num_cores, info.num_subcores
    NW = NC * NS
    assert N % NW == 0
    per_w = N // NW
    C = 256
    assert per_w % (2 * C) == 0
    n_pairs = per_w // (2 * C)
    mesh = plsc.VectorSubcoreMesh(core_axis_name="c", subcore_axis_name="s")

    @functools.partial(
        pl.kernel,
        mesh=mesh,
        out_type=jax.ShapeDtypeStruct((N, D), jnp.float32),
        scratch_types=[
            pltpu.VMEM((per_w,), jnp.int32),
            pltpu.VMEM((C, D), jnp.float32),
            pltpu.VMEM((C, D), jnp.float32),
            pltpu.SemaphoreType.DMA,
            pltpu.SemaphoreType.DMA,
            pltpu.SemaphoreType.DMA,
            pltpu.SemaphoreType.DMA,
        ],
    )
    def gather_k(tab_hbm, idx_hbm, out_hbm, idx_v, r0, r1, sg0, sg1, sw0, sw1):
        wid = lax.axis_index("s") * NC + lax.axis_index("c")
        base = wid * per_w
        pltpu.sync_copy(idx_hbm.at[pl.ds(base, per_w)], idx_v)

        rows = (r0, r1)
        sgs = (sg0, sg1)
        sws = (sw0, sw1)

        def issue_gather(i, b):
            pltpu.async_copy(
                tab_hbm.at[idx_v.at[pl.ds(i * C, C)]], rows[b], sgs[b])

        def wait_gather(b):
            pltpu.make_async_copy(
                tab_hbm.at[idx_v.at[pl.ds(0, C)]], rows[b], sgs[b]).wait()

        def issue_write(i, b):
            pltpu.async_copy(rows[b], out_hbm.at[pl.ds(base + i * C, C)], sws[b])

        def wait_write(b):
            pltpu.make_async_copy(rows[b], out_hbm.at[pl.ds(0, C)], sws[b]).wait()

        issue_gather(0, 0)
        issue_gather(1, 1)

        def body(p, carry):
            i0 = 2 * p
            wait_gather(0)
            issue_write(i0, 0)
            wait_gather(1)
            issue_write(i0 + 1, 1)
            wait_write(0)
            issue_gather(i0 + 2, 0)
            wait_write(1)
            issue_gather(i0 + 3, 1)
            return carry

        lax.fori_loop(0, n_pairs - 1, body, 0)
        i0 = 2 * (n_pairs - 1)
        wait_gather(0)
        issue_write(i0, 0)
        wait_gather(1)
        issue_write(i0 + 1, 1)
        wait_write(0)
        wait_write(1)

    return gather_k


def kernel(x, table, gamma, beta):
    B, L = x.shape
    V, D = table.shape
    norm = _normalize_table(table, gamma, beta)
    flat = x.reshape(-1).astype(jnp.int32)
    out = _make_gather(V, D, B * L)(norm, flat)
    return out.reshape(B, L, D)


# speedup vs baseline: 9.5575x; 1.0075x over previous
import functools

import jax
import jax.numpy as jnp
from jax import lax
from jax.experimental import pallas as pl
from jax.experimental.pallas import tpu as pltpu
from jax.experimental.pallas import tpu_sc as plsc

EPS = 1e-5


def _norm_body(tab_ref, gamma_ref, beta_ref, out_ref):
    xv = tab_ref[...]
    mean = jnp.mean(xv, axis=-1, keepdims=True)
    xc = xv - mean
    var = jnp.mean(xc * xc, axis=-1, keepdims=True)
    out_ref[...] = xc * lax.rsqrt(var + EPS) * gamma_ref[...] + beta_ref[...]


def _normalize_table(table, gamma, beta):
    V, D = table.shape
    R = 2000
    assert V % R == 0
    return pl.pallas_call(
        _norm_body,
        grid=(V // R,),
        in_specs=[
            pl.BlockSpec((R, D), lambda i: (i, 0)),
            pl.BlockSpec((1, D), lambda i: (0, 0)),
            pl.BlockSpec((1, D), lambda i: (0, 0)),
        ],
        out_specs=pl.BlockSpec((R, D), lambda i: (i, 0)),
        out_shape=jax.ShapeDtypeStruct((V, D), jnp.float32),
    )(table, gamma.reshape(1, D), beta.reshape(1, D))


@functools.lru_cache(maxsize=None)
def _make_gather(V, D, N):
    info = plsc.get_sparse_core_info()
    NC, NS = info.num_cores, info.num_subcores
    NW = NC * NS
    assert N % NW == 0
    per_w = N // NW
    C = 128
    NBUF = 4
    assert per_w % (NBUF * C) == 0
    n_outer = per_w // (NBUF * C)
    mesh = plsc.VectorSubcoreMesh(core_axis_name="c", subcore_axis_name="s")

    @functools.partial(
        pl.kernel,
        mesh=mesh,
        out_type=jax.ShapeDtypeStruct((N, D), jnp.float32),
        scratch_types=[
            pltpu.VMEM((per_w,), jnp.int32),
        ] + [pltpu.VMEM((C, D), jnp.float32)] * NBUF
          + [pltpu.SemaphoreType.DMA] * (2 * NBUF),
    )
    def gather_k(tab_hbm, idx_hbm, out_hbm, idx_v, *bufs_and_sems):
        rows = bufs_and_sems[:NBUF]
        sgs = bufs_and_sems[NBUF:2 * NBUF]
        sws = bufs_and_sems[2 * NBUF:]
        wid = lax.axis_index("s") * NC + lax.axis_index("c")
        base = wid * per_w
        pltpu.sync_copy(idx_hbm.at[pl.ds(base, per_w)], idx_v)

        def issue_gather(i, b):
            pltpu.async_copy(
                tab_hbm.at[idx_v.at[pl.ds(i * C, C)]], rows[b], sgs[b])

        def wait_gather(b):
            pltpu.make_async_copy(
                tab_hbm.at[idx_v.at[pl.ds(0, C)]], rows[b], sgs[b]).wait()

        def issue_write(i, b):
            pltpu.async_copy(rows[b], out_hbm.at[pl.ds(base + i * C, C)], sws[b])

        def wait_write(b):
            pltpu.make_async_copy(rows[b], out_hbm.at[pl.ds(0, C)], sws[b]).wait()

        for b in range(NBUF):
            issue_gather(b, b)

        def body(t, carry):
            i0 = t * NBUF
            for b in range(NBUF):
                wait_gather(b)
                issue_write(i0 + b, b)
            for b in range(NBUF):
                wait_write(b)
                issue_gather(i0 + NBUF + b, b)
            return carry

        lax.fori_loop(0, n_outer - 1, body, 0)
        i0 = (n_outer - 1) * NBUF
        for b in range(NBUF):
            wait_gather(b)
            issue_write(i0 + b, b)
        for b in range(NBUF):
            wait_write(b)

    return gather_k


def kernel(x, table, gamma, beta):
    B, L = x.shape
    V, D = table.shape
    norm = _normalize_table(table, gamma, beta)
    flat = x.reshape(-1).astype(jnp.int32)
    out = _make_gather(V, D, B * L)(norm, flat)
    return out.reshape(B, L, D)
